# baseline (speedup 1.0000x reference)
import functools

import jax
import jax.numpy as jnp
from jax import lax
from jax.experimental import pallas as pl
from jax.experimental.pallas import tpu as pltpu
from jax.experimental.pallas import tpu_sc as plsc

N = 10000
E = 320000
D = 128
NG = 64

NUM_SC = 2
NUM_TILES = 16
NW = NUM_SC * NUM_TILES
CH = 128
NBUF = 3
NCHUNK = 81
EPW = CH * NCHUNK
EPAD = EPW * NW
ROWS_PER_TILE = 624
ROWS_REM = N - ROWS_PER_TILE * NUM_TILES


def _h_body(x_ref, w1_ref, b1_ref, h_ref):
    h_ref[...] = (
        jnp.dot(x_ref[...], w1_ref[...], preferred_element_type=jnp.float32)
        + b1_ref[...][None, :]
    )


def _compute_h(x, W1, b1):
    return pl.pallas_call(
        _h_body,
        out_shape=jax.ShapeDtypeStruct((N, D), jnp.float32),
    )(x, W1, b1)


EBLK = E


WBLK = 8192


def _w_body(ea_ref, we_ref, w_ref):
    g = jnp.exp(-(ea_ref[...] ** 2))
    w = jnp.dot(g, we_ref[...], preferred_element_type=jnp.float32)
    w_ref[...] = w.reshape(WBLK // 128, 128)


def _compute_w(edge_attr, We):
    w2d = pl.pallas_call(
        _w_body,
        grid=((E + WBLK - 1) // WBLK,),
        in_specs=[
            pl.BlockSpec((WBLK, 4), lambda i: (i, 0)),
            pl.BlockSpec((4, 1), lambda i: (0, 0)),
        ],
        out_specs=pl.BlockSpec((WBLK // 128, 128), lambda i: (i, 0)),
        out_shape=jax.ShapeDtypeStruct((E // 128, 128), jnp.float32),
    )(edge_attr, We)
    return w2d.reshape(E)


def _sc_body(h_hbm, src_hbm, dst_hbm, w_hbm, zeros_hbm, out_hbm,
             srcs, dsts, ws, rows, acc_sh, sem_i, sem_g, sem_s):
    c = lax.axis_index("c")
    s = lax.axis_index("s")
    wid = c * NUM_TILES + s
    ebase = wid * EPW

    pltpu.sync_copy(
        zeros_hbm.at[pl.ds(s * ROWS_PER_TILE, ROWS_PER_TILE)],
        acc_sh.at[pl.ds(s * ROWS_PER_TILE, ROWS_PER_TILE)],
    )

    @pl.when(s == 0)
    def _():
        pltpu.sync_copy(
            zeros_hbm.at[pl.ds(NUM_TILES * ROWS_PER_TILE, ROWS_REM)],
            acc_sh.at[pl.ds(NUM_TILES * ROWS_PER_TILE, ROWS_REM)],
        )

    def start_idx(i, b):
        base = ebase + i * CH
        pltpu.async_copy(src_hbm.at[pl.ds(base, CH)], srcs[b], sem_i[b])
        pltpu.async_copy(dst_hbm.at[pl.ds(base, CH)], dsts[b], sem_i[b])
        pltpu.async_copy(w_hbm.at[pl.ds(base, CH)], ws[b], sem_i[b])

    def wait_idx(b):
        pltpu.make_async_copy(src_hbm.at[pl.ds(0, CH)], srcs[b],
                              sem_i[b]).wait()
        pltpu.make_async_copy(dst_hbm.at[pl.ds(0, CH)], dsts[b],
                              sem_i[b]).wait()
        pltpu.make_async_copy(w_hbm.at[pl.ds(0, CH)], ws[b], sem_i[b]).wait()

    def start_gather(b):
        pltpu.async_copy(h_hbm.at[srcs[b]], rows[b], sem_g[b])

    def wait_gather(b):
        pltpu.make_async_copy(h_hbm.at[srcs[b]], rows[b], sem_g[b]).wait()

    def start_scatter(b):
        pltpu.async_copy(rows[b], acc_sh.at[dsts[b]], sem_s[b], add=True)

    def wait_scatter(b):
        pltpu.make_async_copy(rows[b], acc_sh.at[dsts[b]], sem_s[b]).wait()

    start_idx(0, 0)
    start_idx(1, 1)
    wait_idx(0)
    start_gather(0)

    def outer(o, carry):
        for k in range(NBUF):
            i = o * NBUF + k
            nb = (k + 1) % NBUF
            n2 = (k + 2) % NBUF

            @pl.when(i + 1 < NCHUNK)
            def _():
                wait_idx(nb)
                start_gather(nb)

            wait_gather(k)

            def edge(e, carry2):
                wv = plsc.load_gather(ws[k], [jnp.full((16,), 0, jnp.int32) + e])
                for f in range(D // 16):
                    sl = pl.ds(f * 16, 16)
                    rows[k][e, sl] = rows[k][e, sl] * wv
                return carry2

            lax.fori_loop(0, CH, edge, 0, unroll=4)
            start_scatter(k)

            @pl.when(i + 2 < NCHUNK)
            def _():
                @pl.when(i + 2 >= NBUF)
                def _():
                    wait_scatter(n2)
                start_idx(i + 2, n2)
        return carry

    lax.fori_loop(0, NCHUNK // NBUF, outer, 0)
    for k in range(NBUF):
        wait_scatter(k)

    plsc.subcore_barrier()
    pltpu.sync_copy(
        acc_sh.at[pl.ds(s * ROWS_PER_TILE, ROWS_PER_TILE)],
        out_hbm.at[c, pl.ds(s * ROWS_PER_TILE, ROWS_PER_TILE)],
    )

    @pl.when(s == 0)
    def _():
        pltpu.sync_copy(
            acc_sh.at[pl.ds(NUM_TILES * ROWS_PER_TILE, ROWS_REM)],
            out_hbm.at[c, pl.ds(NUM_TILES * ROWS_PER_TILE, ROWS_REM)],
        )


def _sc_entry(h_hbm, src_hbm, dst_hbm, w_hbm, zeros_hbm, out_hbm,
              sr0, sr1, sr2, ds0, ds1, ds2, w0, w1, w2,
              r0, r1, r2, acc_sh,
              i0, i1, i2, g0, g1, g2, s0, s1, s2):
    _sc_body(h_hbm, src_hbm, dst_hbm, w_hbm, zeros_hbm, out_hbm,
             [sr0, sr1, sr2], [ds0, ds1, ds2], [w0, w1, w2],
             [r0, r1, r2], acc_sh,
             [i0, i1, i2], [g0, g1, g2], [s0, s1, s2])


@functools.cache
def _sc_aggregate_fn():
    return pl.kernel(
        _sc_entry,
        out_type=jax.ShapeDtypeStruct((NUM_SC, N, D), jnp.float32),
        mesh=plsc.VectorSubcoreMesh(core_axis_name="c", subcore_axis_name="s",
                                    num_cores=NUM_SC, num_subcores=NUM_TILES),
        scratch_types=(
            [pltpu.VMEM((CH,), jnp.int32) for _ in range(NBUF)]
            + [pltpu.VMEM((CH,), jnp.int32) for _ in range(NBUF)]
            + [pltpu.VMEM((CH,), jnp.float32) for _ in range(NBUF)]
            + [pltpu.VMEM((CH, D), jnp.float32) for _ in range(NBUF)]
            + [pltpu.VMEM_SHARED((N, D), jnp.float32)]
            + [pltpu.SemaphoreType.DMA for _ in range(3 * NBUF)]
        ),
        compiler_params=pltpu.CompilerParams(needs_layout_passes=False),
    )


def _sc_aggregate(h, src, dst, w, zeros):
    return _sc_aggregate_fn()(h, src, dst, w, zeros)


def _tail_body(acc_ref, h_ref, batch_ref, gfeat_ref, wc_ref, bc_ref, out_ref):
    agg = acc_ref[0] + acc_ref[1]
    ge = jnp.maximum(agg + h_ref[...], 0.0)
    ids = lax.broadcasted_iota(jnp.int32, (N, NG), 1)
    onehot = (batch_ref[...][:, None] == ids).astype(jnp.float32)
    pooled = lax.dot_general(
        onehot, ge, (((0,), (0,)), ((), ())),
        preferred_element_type=jnp.float32,
    )
    wc = wc_ref[...]
    head = lax.dot_general(
        pooled, wc[:D, :], (((1,), (0,)), ((), ())),
        preferred_element_type=jnp.float32,
    )
    gpart = jnp.dot(gfeat_ref[...], wc[D:, :],
                    preferred_element_type=jnp.float32)
    out_ref[...] = head + gpart + bc_ref[...][None, :]


def _tail(acc2, h, batch, gfeat, Wc, bc):
    nc = Wc.shape[1]
    return pl.pallas_call(
        _tail_body,
        out_shape=jax.ShapeDtypeStruct((NG, nc), jnp.float32),
    )(acc2, h, batch, gfeat, Wc, bc)


def kernel(x, edge_attr, W1, b1, We, gfeat, Wc, bc, edge_index, batch):
    src = edge_index[0].astype(jnp.int32)
    dst = edge_index[1].astype(jnp.int32)
    h = _compute_h(x, W1, b1)
    w = _compute_w(edge_attr, We)
    pad_idx = jnp.arange(EPAD - E, dtype=jnp.int32) % N
    src_p = jnp.concatenate([src, pad_idx])
    dst_p = jnp.concatenate([dst, pad_idx])
    w_p = jnp.concatenate([w, jnp.zeros((EPAD - E,), jnp.float32)])
    zeros = jnp.zeros((N, D), jnp.float32)
    acc2 = _sc_aggregate(h, src_p, dst_p, w_p, zeros)
    return _tail(acc2, h, batch, gfeat, Wc, bc)


# speedup vs baseline: 1.0360x; 1.0360x over previous
import functools

import jax
import jax.numpy as jnp
from jax import lax
from jax.experimental import pallas as pl
from jax.experimental.pallas import tpu as pltpu
from jax.experimental.pallas import tpu_sc as plsc

N = 10000
E = 320000
D = 128
NG = 64

NUM_SC = 2
NUM_TILES = 16
NW = NUM_SC * NUM_TILES
CH = 128
NBUF = 3
NCHUNK = 81
EPW = CH * NCHUNK
EPAD = EPW * NW
ROWS_PER_TILE = 624
ROWS_REM = N - ROWS_PER_TILE * NUM_TILES


def _h_body(x_ref, w1_ref, b1_ref, h_ref):
    h_ref[...] = (
        jnp.dot(x_ref[...], w1_ref[...], preferred_element_type=jnp.float32)
        + b1_ref[...][None, :]
    )


def _compute_h(x, W1, b1):
    return pl.pallas_call(
        _h_body,
        out_shape=jax.ShapeDtypeStruct((N, D), jnp.float32),
    )(x, W1, b1)


EBLK = E


WBLK = 16384


def _w_body(ea_ref, we_ref, w_ref):
    g = jnp.exp(-(ea_ref[...] ** 2))
    w = jnp.dot(g, we_ref[...], preferred_element_type=jnp.float32)
    w_ref[...] = w.reshape(WBLK // 128, 128)


def _compute_w(edge_attr, We):
    w2d = pl.pallas_call(
        _w_body,
        grid=((E + WBLK - 1) // WBLK,),
        in_specs=[
            pl.BlockSpec((WBLK, 4), lambda i: (i, 0)),
            pl.BlockSpec((4, 1), lambda i: (0, 0)),
        ],
        out_specs=pl.BlockSpec((WBLK // 128, 128), lambda i: (i, 0)),
        out_shape=jax.ShapeDtypeStruct((E // 128, 128), jnp.float32),
    )(edge_attr, We)
    return w2d.reshape(E)


def _sc_body(h_hbm, src_hbm, dst_hbm, w_hbm, zeros_hbm, out_hbm,
             srcs, dsts, ws, rows, acc_sh, sem_i, sem_g, sem_s):
    c = lax.axis_index("c")
    s = lax.axis_index("s")
    wid = c * NUM_TILES + s
    ebase = wid * EPW

    pltpu.sync_copy(
        zeros_hbm.at[pl.ds(s * ROWS_PER_TILE, ROWS_PER_TILE)],
        acc_sh.at[pl.ds(s * ROWS_PER_TILE, ROWS_PER_TILE)],
    )

    @pl.when(s == 0)
    def _():
        pltpu.sync_copy(
            zeros_hbm.at[pl.ds(NUM_TILES * ROWS_PER_TILE, ROWS_REM)],
            acc_sh.at[pl.ds(NUM_TILES * ROWS_PER_TILE, ROWS_REM)],
        )

    def start_idx(i, b):
        base = ebase + i * CH
        pltpu.async_copy(src_hbm.at[pl.ds(base, CH)], srcs[b], sem_i[b])
        pltpu.async_copy(dst_hbm.at[pl.ds(base, CH)], dsts[b], sem_i[b])
        pltpu.async_copy(w_hbm.at[pl.ds(base, CH)], ws[b], sem_i[b])

    def wait_idx(b):
        pltpu.make_async_copy(src_hbm.at[pl.ds(0, CH)], srcs[b],
                              sem_i[b]).wait()
        pltpu.make_async_copy(dst_hbm.at[pl.ds(0, CH)], dsts[b],
                              sem_i[b]).wait()
        pltpu.make_async_copy(w_hbm.at[pl.ds(0, CH)], ws[b], sem_i[b]).wait()

    def start_gather(b):
        pltpu.async_copy(h_hbm.at[srcs[b]], rows[b], sem_g[b])

    def wait_gather(b):
        pltpu.make_async_copy(h_hbm.at[srcs[b]], rows[b], sem_g[b]).wait()

    def start_scatter(b):
        pltpu.async_copy(rows[b], acc_sh.at[dsts[b]], sem_s[b], add=True)

    def wait_scatter(b):
        pltpu.make_async_copy(rows[b], acc_sh.at[dsts[b]], sem_s[b]).wait()

    start_idx(0, 0)
    start_idx(1, 1)
    wait_idx(0)
    start_gather(0)

    def outer(o, carry):
        for k in range(NBUF):
            i = o * NBUF + k
            nb = (k + 1) % NBUF
            n2 = (k + 2) % NBUF

            @pl.when(i + 1 < NCHUNK)
            def _():
                wait_idx(nb)
                start_gather(nb)

            wait_gather(k)

            def edge(e, carry2):
                wv = plsc.load_gather(ws[k], [jnp.full((16,), 0, jnp.int32) + e])
                for f in range(D // 16):
                    sl = pl.ds(f * 16, 16)
                    rows[k][e, sl] = rows[k][e, sl] * wv
                return carry2

            lax.fori_loop(0, CH, edge, 0, unroll=4)
            start_scatter(k)

            @pl.when(i + 2 < NCHUNK)
            def _():
                @pl.when(i + 2 >= NBUF)
                def _():
                    wait_scatter(n2)
                start_idx(i + 2, n2)
        return carry

    lax.fori_loop(0, NCHUNK // NBUF, outer, 0)
    for k in range(NBUF):
        wait_scatter(k)

    plsc.subcore_barrier()
    pltpu.sync_copy(
        acc_sh.at[pl.ds(s * ROWS_PER_TILE, ROWS_PER_TILE)],
        out_hbm.at[c, pl.ds(s * ROWS_PER_TILE, ROWS_PER_TILE)],
    )

    @pl.when(s == 0)
    def _():
        pltpu.sync_copy(
            acc_sh.at[pl.ds(NUM_TILES * ROWS_PER_TILE, ROWS_REM)],
            out_hbm.at[c, pl.ds(NUM_TILES * ROWS_PER_TILE, ROWS_REM)],
        )


def _sc_entry(h_hbm, src_hbm, dst_hbm, w_hbm, zeros_hbm, out_hbm,
              sr0, sr1, sr2, ds0, ds1, ds2, w0, w1, w2,
              r0, r1, r2, acc_sh,
              i0, i1, i2, g0, g1, g2, s0, s1, s2):
    _sc_body(h_hbm, src_hbm, dst_hbm, w_hbm, zeros_hbm, out_hbm,
             [sr0, sr1, sr2], [ds0, ds1, ds2], [w0, w1, w2],
             [r0, r1, r2], acc_sh,
             [i0, i1, i2], [g0, g1, g2], [s0, s1, s2])


@functools.cache
def _sc_aggregate_fn():
    return pl.kernel(
        _sc_entry,
        out_type=jax.ShapeDtypeStruct((NUM_SC, N, D), jnp.float32),
        mesh=plsc.VectorSubcoreMesh(core_axis_name="c", subcore_axis_name="s",
                                    num_cores=NUM_SC, num_subcores=NUM_TILES),
        scratch_types=(
            [pltpu.VMEM((CH,), jnp.int32) for _ in range(NBUF)]
            + [pltpu.VMEM((CH,), jnp.int32) for _ in range(NBUF)]
            + [pltpu.VMEM((CH,), jnp.float32) for _ in range(NBUF)]
            + [pltpu.VMEM((CH, D), jnp.float32) for _ in range(NBUF)]
            + [pltpu.VMEM_SHARED((N, D), jnp.float32)]
            + [pltpu.SemaphoreType.DMA for _ in range(3 * NBUF)]
        ),
        compiler_params=pltpu.CompilerParams(needs_layout_passes=False),
    )


def _sc_aggregate(h, src, dst, w, zeros):
    return _sc_aggregate_fn()(h, src, dst, w, zeros)


def _tail_body(acc_ref, h_ref, batch_ref, gfeat_ref, wc_ref, bc_ref, out_ref):
    agg = acc_ref[0] + acc_ref[1]
    ge = jnp.maximum(agg + h_ref[...], 0.0)
    ids = lax.broadcasted_iota(jnp.int32, (N, NG), 1)
    onehot = (batch_ref[...][:, None] == ids).astype(jnp.float32)
    pooled = lax.dot_general(
        onehot, ge, (((0,), (0,)), ((), ())),
        preferred_element_type=jnp.float32,
    )
    wc = wc_ref[...]
    head = lax.dot_general(
        pooled, wc[:D, :], (((1,), (0,)), ((), ())),
        preferred_element_type=jnp.float32,
    )
    gpart = jnp.dot(gfeat_ref[...], wc[D:, :],
                    preferred_element_type=jnp.float32)
    out_ref[...] = head + gpart + bc_ref[...][None, :]


def _tail(acc2, h, batch, gfeat, Wc, bc):
    nc = Wc.shape[1]
    return pl.pallas_call(
        _tail_body,
        out_shape=jax.ShapeDtypeStruct((NG, nc), jnp.float32),
    )(acc2, h, batch, gfeat, Wc, bc)


def kernel(x, edge_attr, W1, b1, We, gfeat, Wc, bc, edge_index, batch):
    src = edge_index[0].astype(jnp.int32)
    dst = edge_index[1].astype(jnp.int32)
    h = _compute_h(x, W1, b1)
    w = _compute_w(edge_attr, We)
    pad_idx = jnp.arange(EPAD - E, dtype=jnp.int32) % N
    src_p = jnp.concatenate([src, pad_idx])
    dst_p = jnp.concatenate([dst, pad_idx])
    w_p = jnp.concatenate([w, jnp.zeros((EPAD - E,), jnp.float32)])
    zeros = jnp.zeros((N, D), jnp.float32)
    acc2 = _sc_aggregate(h, src_p, dst_p, w_p, zeros)
    return _tail(acc2, h, batch, gfeat, Wc, bc)
